# baseline (speedup 1.0000x reference)
import functools

import jax
import jax.numpy as jnp
from jax import lax
from jax.experimental import pallas as pl
from jax.experimental.pallas import tpu as pltpu
from jax.experimental.pallas import tpu_sc as plsc

_B = 16384
_NW = 32
_BPW = _B // _NW


def _sc_gather_body(song_id_h, art_id_h, gen_id_h, lang_id_h,
                    song_t, art_t, gen_t, lang_t,
                    song_o, art_o, gen_o, lang_o,
                    sidx, aidx, gidx, lidx,
                    srows, arows, grows, lrows, sem):
  wid = lax.axis_index("s") * 2 + lax.axis_index("c")
  base = wid * _BPW
  pltpu.sync_copy(song_id_h.at[pl.ds(base, _BPW)], sidx)
  pltpu.sync_copy(art_id_h.at[pl.ds(base, _BPW)], aidx)
  pltpu.sync_copy(gen_id_h.at[pl.ds(base, _BPW)], gidx)
  pltpu.sync_copy(lang_id_h.at[pl.ds(base, _BPW)], lidx)
  c1 = pltpu.async_copy(song_t.at[sidx], srows, sem)
  c2 = pltpu.async_copy(art_t.at[aidx], arows, sem)
  c3 = pltpu.async_copy(gen_t.at[gidx], grows, sem)
  c4 = pltpu.async_copy(lang_t.at[lidx], lrows, sem)
  c1.wait()
  c2.wait()
  c3.wait()
  c4.wait()
  pltpu.sync_copy(srows, song_o.at[pl.ds(base, _BPW)])
  pltpu.sync_copy(arows, art_o.at[pl.ds(base, _BPW)])
  pltpu.sync_copy(grows, gen_o.at[pl.ds(base, _BPW)])
  pltpu.sync_copy(lrows, lang_o.at[pl.ds(base, _BPW)])


_sc_gather = pl.kernel(
    _sc_gather_body,
    out_type=[
        jax.ShapeDtypeStruct((_B, 64), jnp.float32),
        jax.ShapeDtypeStruct((_B, 64), jnp.float32),
        jax.ShapeDtypeStruct((_B, 32), jnp.float32),
        jax.ShapeDtypeStruct((_B, 32), jnp.float32),
    ],
    mesh=plsc.VectorSubcoreMesh(core_axis_name="c", subcore_axis_name="s"),
    scratch_types=[
        pltpu.VMEM((_BPW,), jnp.int32),
        pltpu.VMEM((_BPW,), jnp.int32),
        pltpu.VMEM((_BPW,), jnp.int32),
        pltpu.VMEM((_BPW,), jnp.int32),
        pltpu.VMEM((_BPW, 64), jnp.float32),
        pltpu.VMEM((_BPW, 64), jnp.float32),
        pltpu.VMEM((_BPW, 32), jnp.float32),
        pltpu.VMEM((_BPW, 32), jnp.float32),
        pltpu.SemaphoreType.DMA,
    ],
    compiler_params=pltpu.CompilerParams(use_tc_tiling_on_sc=False),
    name="sc_embedding_gather",
)


_CHUNK = 2048


def _mlp_body(song, art, gen, lang, num,
              w1a, w1b, w1c, w1d, wnum, bnum, w1e, b1, w2, b2, w3, b3, out):
  acc = jnp.dot(song[...], w1a[...], preferred_element_type=jnp.float32)
  acc += jnp.dot(art[...], w1b[...], preferred_element_type=jnp.float32)
  acc += jnp.dot(gen[...], w1c[...], preferred_element_type=jnp.float32)
  acc += jnp.dot(lang[...], w1d[...], preferred_element_type=jnp.float32)
  nv = jnp.dot(num[...], wnum[...], preferred_element_type=jnp.float32) + bnum[...]
  acc += jnp.dot(nv, w1e[...], preferred_element_type=jnp.float32)
  h1 = jnp.maximum(acc + b1[...], 0.0)
  h2 = jnp.maximum(jnp.dot(h1, w2[...], preferred_element_type=jnp.float32) + b2[...], 0.0)
  out[...] = jnp.dot(h2, w3[...], preferred_element_type=jnp.float32) + b3[...]


def _tc_mlp(song_emb, art_emb, gen_emb, lang_emb, num,
            w1a, w1b, w1c, w1d, wnum, bnum, w1e, b1, w2, b2, w3, b3):
  nsteps = _B // _CHUNK
  row_spec = lambda width: pl.BlockSpec((_CHUNK, width), lambda i: (i, 0))
  full = lambda a: pl.BlockSpec(a.shape, lambda i: (0,) * a.ndim)
  return pl.pallas_call(
      _mlp_body,
      grid=(nsteps,),
      in_specs=[
          row_spec(64), row_spec(64), row_spec(32), row_spec(32), row_spec(8),
          full(w1a), full(w1b), full(w1c), full(w1d), full(wnum), full(bnum),
          full(w1e), full(b1), full(w2), full(b2), full(w3), full(b3),
      ],
      out_specs=pl.BlockSpec((_CHUNK, 64), lambda i: (i, 0)),
      out_shape=jax.ShapeDtypeStruct((_B, 64), jnp.float32),
  )(song_emb, art_emb, gen_emb, lang_emb, num,
    w1a, w1b, w1c, w1d, wnum, bnum, w1e, b1, w2, b2, w3, b3)


def kernel(song_id, artist_encoded, genre_encoded, language_encoded,
           numerical_features, song_table, artist_table, genre_table,
           language_table, W_num, b_num, W1, b1, W2, b2, W3, b3):
  gen_t = jnp.pad(genre_table, ((0, 0), (0, 1)))
  lang_t = jnp.pad(language_table, ((0, 0), (0, 1)))
  song_emb, art_emb, gen_emb, lang_emb = _sc_gather(
      song_id.astype(jnp.int32), artist_encoded.astype(jnp.int32),
      genre_encoded.astype(jnp.int32), language_encoded.astype(jnp.int32),
      song_table, artist_table, gen_t, lang_t)
  w1a = W1[0:64]
  w1b = W1[64:128]
  w1c = jnp.pad(W1[128:159], ((0, 1), (0, 0)))
  w1d = jnp.pad(W1[159:190], ((0, 1), (0, 0)))
  w1e = W1[190:206]
  return _tc_mlp(song_emb, art_emb, gen_emb, lang_emb, numerical_features,
                 w1a, w1b, w1c, w1d, W_num, b_num.reshape(1, 16), w1e,
                 b1.reshape(1, 256), W2, b2.reshape(1, 128), W3,
                 b3.reshape(1, 64))


# speedup vs baseline: 1.1061x; 1.1061x over previous
import jax
import jax.numpy as jnp
from jax import lax
from jax.experimental import pallas as pl
from jax.experimental.pallas import tpu as pltpu
from jax.experimental.pallas import tpu_sc as plsc

_B = 16384
_NW = 32
_BPW = _B // _NW
_PW = 128


_HB = _BPW // 2


def _sc_gather_body(song_id_h, art_id_h, gen_id_h, lang_id_h,
                    song_t, art_t, gen_t, lang_t,
                    song_o, art_o, gen_o, lang_o,
                    sidx, aidx, gidx, lidx, rows_a, rows_b, sem):
  wid = lax.axis_index("s") * 2 + lax.axis_index("c")
  base = wid * _BPW
  pltpu.sync_copy(song_id_h.at[pl.ds(base, _BPW)], sidx)
  pltpu.sync_copy(art_id_h.at[pl.ds(base, _BPW)], aidx)
  pltpu.sync_copy(gen_id_h.at[pl.ds(base, _BPW)], gidx)
  pltpu.sync_copy(lang_id_h.at[pl.ds(base, _BPW)], lidx)
  tasks = []
  for tab, idx_v, out in ((song_t, sidx, song_o), (art_t, aidx, art_o),
                          (gen_t, gidx, gen_o), (lang_t, lidx, lang_o)):
    for h in (0, 1):
      tasks.append((tab, idx_v, out, h))
  bufs = (rows_a, rows_b)
  copies = [None] * len(tasks)
  for k in (0, 1):
    tab, idx_v, _, h = tasks[k]
    copies[k] = pltpu.async_copy(tab.at[idx_v.at[pl.ds(h * _HB, _HB)]],
                                 bufs[k % 2], sem)
  for k, (tab, idx_v, out, h) in enumerate(tasks):
    copies[k].wait()
    pltpu.sync_copy(bufs[k % 2], out.at[pl.ds(base + h * _HB, _HB)])
    if k + 2 < len(tasks):
      tab2, idx2, _, h2 = tasks[k + 2]
      copies[k + 2] = pltpu.async_copy(
          tab2.at[idx2.at[pl.ds(h2 * _HB, _HB)]], bufs[k % 2], sem)


_sc_gather = pl.kernel(
    _sc_gather_body,
    out_type=[jax.ShapeDtypeStruct((_B, _PW), jnp.float32)] * 4,
    mesh=plsc.VectorSubcoreMesh(core_axis_name="c", subcore_axis_name="s"),
    scratch_types=[
        pltpu.VMEM((_BPW,), jnp.int32),
        pltpu.VMEM((_BPW,), jnp.int32),
        pltpu.VMEM((_BPW,), jnp.int32),
        pltpu.VMEM((_BPW,), jnp.int32),
        pltpu.VMEM((_HB, _PW), jnp.float32),
        pltpu.VMEM((_HB, _PW), jnp.float32),
        pltpu.SemaphoreType.DMA,
    ],
    compiler_params=pltpu.CompilerParams(use_tc_tiling_on_sc=True),
    name="sc_embedding_gather",
)


_CHUNK = 2048


def _mlp_body(song, art, gen, lang, num,
              w1a, w1b, w1c, w1d, wnum, bnum, w1e, b1, w2, b2, w3, b3, out):
  acc = jnp.dot(song[...], w1a[...], preferred_element_type=jnp.float32)
  acc += jnp.dot(art[...], w1b[...], preferred_element_type=jnp.float32)
  acc += jnp.dot(gen[...], w1c[...], preferred_element_type=jnp.float32)
  acc += jnp.dot(lang[...], w1d[...], preferred_element_type=jnp.float32)
  nv = jnp.dot(num[...], wnum[...], preferred_element_type=jnp.float32) + bnum[...]
  acc += jnp.dot(nv, w1e[...], preferred_element_type=jnp.float32)
  h1 = jnp.maximum(acc + b1[...], 0.0)
  h2 = jnp.maximum(jnp.dot(h1, w2[...], preferred_element_type=jnp.float32) + b2[...], 0.0)
  out[...] = jnp.dot(h2, w3[...], preferred_element_type=jnp.float32) + b3[...]


def _tc_mlp(song_emb, art_emb, gen_emb, lang_emb, num,
            w1a, w1b, w1c, w1d, wnum, bnum, w1e, b1, w2, b2, w3, b3):
  nsteps = _B // _CHUNK
  row_spec = lambda width: pl.BlockSpec((_CHUNK, width), lambda i: (i, 0))
  full = lambda a: pl.BlockSpec(a.shape, lambda i: (0,) * a.ndim)
  return pl.pallas_call(
      _mlp_body,
      grid=(nsteps,),
      in_specs=[
          row_spec(_PW), row_spec(_PW), row_spec(_PW), row_spec(_PW),
          row_spec(8),
          full(w1a), full(w1b), full(w1c), full(w1d), full(wnum), full(bnum),
          full(w1e), full(b1), full(w2), full(b2), full(w3), full(b3),
      ],
      out_specs=pl.BlockSpec((_CHUNK, 64), lambda i: (i, 0)),
      out_shape=jax.ShapeDtypeStruct((_B, 64), jnp.float32),
  )(song_emb, art_emb, gen_emb, lang_emb, num,
    w1a, w1b, w1c, w1d, wnum, bnum, w1e, b1, w2, b2, w3, b3)


def kernel(song_id, artist_encoded, genre_encoded, language_encoded,
           numerical_features, song_table, artist_table, genre_table,
           language_table, W_num, b_num, W1, b1, W2, b2, W3, b3):
  song_p = jnp.pad(song_table, ((0, 0), (0, _PW - 64)))
  art_p = jnp.pad(artist_table, ((0, 0), (0, _PW - 64)))
  gen_p = jnp.pad(genre_table, ((0, 0), (0, _PW - 31)))
  lang_p = jnp.pad(language_table, ((0, 0), (0, _PW - 31)))
  song_emb, art_emb, gen_emb, lang_emb = _sc_gather(
      song_id.astype(jnp.int32), artist_encoded.astype(jnp.int32),
      genre_encoded.astype(jnp.int32), language_encoded.astype(jnp.int32),
      song_p, art_p, gen_p, lang_p)
  w1a = jnp.pad(W1[0:64], ((0, _PW - 64), (0, 0)))
  w1b = jnp.pad(W1[64:128], ((0, _PW - 64), (0, 0)))
  w1c = jnp.pad(W1[128:159], ((0, _PW - 31), (0, 0)))
  w1d = jnp.pad(W1[159:190], ((0, _PW - 31), (0, 0)))
  w1e = W1[190:206]
  return _tc_mlp(song_emb, art_emb, gen_emb, lang_emb, numerical_features,
                 w1a, w1b, w1c, w1d, W_num, b_num.reshape(1, 16), w1e,
                 b1.reshape(1, 256), W2, b2.reshape(1, 128), W3,
                 b3.reshape(1, 64))
